# baseline (device time: 15626 ns/iter reference)
import jax
import jax.numpy as jnp
from jax import lax
from jax.experimental import pallas as pl
from jax.experimental.pallas import tpu as pltpu

N_DEV = 4
N_GLOBAL = 4096
EPS = 1e-5
NB = 8


def kernel(x, gamma):
    m, n_per = x.shape
    rb = m // NB
    pr = m // 128
    gamma2 = gamma.reshape(1, n_per)

    def body(x_hbm, g_ref, out_hbm, xv, ov, src_ref, comm_ref,
             send_sems, recv_sems, in_sems, out_sems):
        my = lax.axis_index("i")

        in_copies = []
        for b in range(NB):
            cp = pltpu.make_async_copy(
                x_hbm.at[pl.ds(b * rb, rb)], xv.at[pl.ds(b * rb, rb)],
                in_sems.at[b],
            )
            cp.start()
            in_copies.append(cp)

        barrier_sem = pltpu.get_barrier_semaphore()
        for d in range(1, N_DEV):
            pl.semaphore_signal(
                barrier_sem,
                inc=1,
                device_id=((my + d) % N_DEV,),
                device_id_type=pl.DeviceIdType.MESH,
            )
        pl.semaphore_wait(barrier_sem, N_DEV - 1)

        pbs = []
        for b in range(NB):
            in_copies[b].wait()
            xb = xv[pl.ds(b * rb, rb), :]
            pbs.append(jnp.sum(xb * xb, axis=1, keepdims=True))
        partial = jnp.concatenate(pbs, axis=0)
        p16 = jnp.reshape(partial, (pr, 128))
        src_ref[...] = p16

        rdmas = []
        for d in range(1, N_DEV):
            rdma = pltpu.make_async_remote_copy(
                src_ref=src_ref,
                dst_ref=comm_ref.at[d - 1],
                send_sem=send_sems.at[d - 1],
                recv_sem=recv_sems.at[d - 1],
                device_id=((my + d) % N_DEV,),
                device_id_type=pl.DeviceIdType.MESH,
            )
            rdma.start()
            rdmas.append(rdma)

        total = p16
        for d in range(1, N_DEV):
            rdmas[d - 1].wait()
            total = total + comm_ref[d - 1]

        inv_rms = jnp.reshape(lax.rsqrt(total / N_GLOBAL + EPS), (m, 1))
        g = g_ref[...]

        out_copies = []
        for b in range(NB):
            rows = pl.ds(b * rb, rb)
            ov[rows, :] = (
                xv[rows, :] * g * inv_rms[b * rb:(b + 1) * rb, :]
            ).astype(jnp.bfloat16)
            cp = pltpu.make_async_copy(
                ov.at[rows], out_hbm.at[rows], out_sems.at[b]
            )
            cp.start()
            out_copies.append(cp)
        for b in range(NB):
            out_copies[b].wait()

    return pl.pallas_call(
        body,
        out_shape=jax.ShapeDtypeStruct((m, n_per), jnp.bfloat16),
        in_specs=[
            pl.BlockSpec(memory_space=pl.ANY),
            pl.BlockSpec(memory_space=pltpu.VMEM),
        ],
        out_specs=pl.BlockSpec(memory_space=pl.ANY),
        scratch_shapes=[
            pltpu.VMEM((m, n_per), jnp.float32),
            pltpu.VMEM((m, n_per), jnp.bfloat16),
            pltpu.VMEM((pr, 128), jnp.float32),
            pltpu.VMEM((N_DEV - 1, pr, 128), jnp.float32),
            pltpu.SemaphoreType.DMA((N_DEV - 1,)),
            pltpu.SemaphoreType.DMA((N_DEV - 1,)),
            pltpu.SemaphoreType.DMA((NB,)),
            pltpu.SemaphoreType.DMA((NB,)),
        ],
        compiler_params=pltpu.CompilerParams(collective_id=0),
    )(x, gamma2)


# device time: 11084 ns/iter; 1.4098x vs baseline; 1.4098x over previous
import jax
import jax.numpy as jnp
from jax import lax
from jax.experimental import pallas as pl
from jax.experimental.pallas import tpu as pltpu

N_DEV = 4
N_GLOBAL = 4096
EPS = 1e-5
NH = 2


def kernel(x, gamma):
    m, n_per = x.shape
    mh = m // NH
    ph = mh // 128
    gamma2 = gamma.reshape(1, n_per)
    x = pltpu.with_memory_space_constraint(x, pltpu.MemorySpace.HBM)
    gamma2 = pltpu.with_memory_space_constraint(gamma2, pltpu.MemorySpace.HBM)

    def body(x_hbm, g_hbm, out_ref, xv, gv, src_ref, comm_ref,
             send_sems, recv_sems, in_sems, g_sem):
        my = lax.axis_index("i")

        g_copy = pltpu.make_async_copy(g_hbm, gv, g_sem)
        g_copy.start()
        in_copies = []
        for h in range(NH):
            cp = pltpu.make_async_copy(
                x_hbm.at[pl.ds(h * mh, mh)], xv.at[pl.ds(h * mh, mh)],
                in_sems.at[h],
            )
            cp.start()
            in_copies.append(cp)

        barrier_sem = pltpu.get_barrier_semaphore()
        for d in range(1, N_DEV):
            pl.semaphore_signal(
                barrier_sem,
                inc=1,
                device_id=((my + d) % N_DEV,),
                device_id_type=pl.DeviceIdType.MESH,
            )

        def pass1_and_send(h):
            in_copies[h].wait()
            xb = xv[pl.ds(h * mh, mh), :]
            pb = jnp.sum(xb * xb, axis=1, keepdims=True)
            p_h = jnp.reshape(pb, (ph, 128))
            src_ref[pl.ds(h * ph, ph), :] = p_h
            if h == 0:
                pl.semaphore_wait(barrier_sem, N_DEV - 1)
            rdmas = []
            for d in range(1, N_DEV):
                rdma = pltpu.make_async_remote_copy(
                    src_ref=src_ref.at[pl.ds(h * ph, ph)],
                    dst_ref=comm_ref.at[h, d - 1],
                    send_sem=send_sems.at[h, d - 1],
                    recv_sem=recv_sems.at[h, d - 1],
                    device_id=((my + d) % N_DEV,),
                    device_id_type=pl.DeviceIdType.MESH,
                )
                rdma.start()
                rdmas.append(rdma)
            return p_h, rdmas

        def pass2(h, chunk):
            p_h, rdmas = chunk
            total = p_h
            for d in range(1, N_DEV):
                rdmas[d - 1].wait()
                total = total + comm_ref[h, d - 1]
            inv_h = jnp.reshape(lax.rsqrt(total / N_GLOBAL + EPS), (mh, 1))
            rows = pl.ds(h * mh, mh)
            out_ref[rows, :] = (
                xv[rows, :].astype(jnp.bfloat16)
                * gv[...].astype(jnp.bfloat16)
                * inv_h.astype(jnp.bfloat16)
            )

        chunks = [None] * NH
        chunks[0] = pass1_and_send(0)
        g_copy.wait()
        for h in range(1, NH):
            chunks[h] = pass1_and_send(h)
            pass2(h - 1, chunks[h - 1])
        pass2(NH - 1, chunks[NH - 1])

    return pl.pallas_call(
        body,
        out_shape=jax.ShapeDtypeStruct((m, n_per), jnp.bfloat16),
        in_specs=[
            pl.BlockSpec(memory_space=pltpu.MemorySpace.HBM),
            pl.BlockSpec(memory_space=pltpu.MemorySpace.HBM),
        ],
        out_specs=pl.BlockSpec(memory_space=pltpu.VMEM),
        scratch_shapes=[
            pltpu.VMEM((m, n_per), jnp.float32),
            pltpu.VMEM((1, n_per), jnp.float32),
            pltpu.VMEM((m // 128, 128), jnp.float32),
            pltpu.VMEM((NH, N_DEV - 1, m // NH // 128, 128), jnp.float32),
            pltpu.SemaphoreType.DMA((NH, N_DEV - 1)),
            pltpu.SemaphoreType.DMA((NH, N_DEV - 1)),
            pltpu.SemaphoreType.DMA((NH,)),
            pltpu.SemaphoreType.DMA,
        ],
        compiler_params=pltpu.CompilerParams(collective_id=0),
    )(x, gamma2)
